# Initial kernel scaffold; baseline (speedup 1.0000x reference)
import jax
import jax.numpy as jnp
from jax.experimental import pallas as pl


def kernel(x, edge_index, batch, params):
    raise NotImplementedError("write your pallas kernel here")


# speedup vs baseline: 1.0269x; 1.0269x over previous
import functools

import jax
import jax.numpy as jnp
from jax.experimental import pallas as pl
from jax.experimental.pallas import tpu as pltpu

_HIGH = jax.lax.Precision.HIGHEST


def _dot(a, b):
    return jnp.dot(a, b, preferred_element_type=jnp.float32, precision=_HIGH)


def _dot_t(a, b):
    return jax.lax.dot_general(a, b, (((0,), (0,)), ((), ())),
                               preferred_element_type=jnp.float32,
                               precision=_HIGH)


def _p0_body(x_ref, w_ref, z_ref):
    z_ref[...] = _dot(x_ref[...], w_ref[...])


def _p1_body(z_ref, agg_ref, e1p_ref, b1_ref, u_ref, st_ref):
    i = pl.program_id(0)
    u = e1p_ref[...] * z_ref[...] + agg_ref[...] + b1_ref[...]
    u_ref[...] = u

    @pl.when(i == 0)
    def _():
        st_ref[...] = jnp.zeros_like(st_ref)

    st_ref[0:1, :] += jnp.sum(u, axis=0, keepdims=True)
    st_ref[1:2, :] += jnp.sum(u * u, axis=0, keepdims=True)


def _bn_scale_shift(st, g, b, n):
    mean = st[0:1, :] / n
    var = st[1:2, :] / n - mean * mean
    scale = g * jax.lax.rsqrt(var + 1e-5)
    return scale, b - mean * scale


def _p2_body(u_ref, st_ref, g_ref, b_ref, w2_ref, b2_ref, v_ref, st2_ref, *,
             n_nodes):
    i = pl.program_id(0)
    scale, shift = _bn_scale_shift(st_ref[...], g_ref[...], b_ref[...], n_nodes)
    r = jnp.maximum(u_ref[...] * scale + shift, 0.0)
    v = _dot(r, w2_ref[...]) + b2_ref[...]
    v_ref[...] = v

    @pl.when(i == 0)
    def _():
        st2_ref[...] = jnp.zeros_like(st2_ref)

    st2_ref[0:1, :] += jnp.sum(v, axis=0, keepdims=True)
    st2_ref[1:2, :] += jnp.sum(v * v, axis=0, keepdims=True)


def _p3_body(v_ref, st_ref, g_ref, b_ref, w1n_ref, zn_ref, *, n_nodes):
    scale, shift = _bn_scale_shift(st_ref[...], g_ref[...], b_ref[...], n_nodes)
    h = jnp.maximum(v_ref[...] * scale + shift, 0.0)
    zn_ref[...] = _dot(h, w1n_ref[...])


def _p3f_body(v_ref, st_ref, g_ref, b_ref, bat_ref, wq1b_ref,
              h_ref, ge2_ref, pool_ref, cnt_ref, *, n_nodes, n_graphs):
    i = pl.program_id(0)
    scale, shift = _bn_scale_shift(st_ref[...], g_ref[...], b_ref[...], n_nodes)
    h = jnp.maximum(v_ref[...] * scale + shift, 0.0)
    h_ref[...] = h

    bids = bat_ref[...]
    lanes = jax.lax.broadcasted_iota(jnp.int32, (bids.shape[0], n_graphs), 1)
    onehot = (bids == lanes).astype(jnp.float32)

    @pl.when(i == 0)
    def _():
        pool_ref[...] = jnp.zeros_like(pool_ref)
        cnt_ref[...] = jnp.zeros_like(cnt_ref)

    pool_ref[...] += _dot_t(onehot, h)
    cnt_ref[...] += _dot_t(onehot, jnp.ones((bids.shape[0], 1), jnp.float32))

    @pl.when(i == pl.num_programs(0) - 1)
    def _():
        cnt = jnp.maximum(cnt_ref[...], 1.0)
        ge = pool_ref[...] * (1.0 / cnt)
        ge2_ref[...] = _dot(ge, wq1b_ref[...])


def _p4_body(h_ref, bat_ref, ge2_ref, wq1t_ref, bq1_ref, wq2_ref, bq2_ref,
             q_ref, *, n_graphs):
    bids = bat_ref[...]
    lanes = jax.lax.broadcasted_iota(jnp.int32, (bids.shape[0], n_graphs), 1)
    onehot = (bids == lanes).astype(jnp.float32)
    c = _dot(h_ref[...], wq1t_ref[...]) + _dot(onehot, ge2_ref[...])
    r = jnp.maximum(c + bq1_ref[...], 0.0)
    q = jnp.sum(r * wq2_ref[...], axis=1, keepdims=True) + bq2_ref[...]
    q_ref[...] = q


def _row_spec(bn, w):
    return pl.BlockSpec((bn, w), lambda i: (i, 0))


def _const_spec(shape):
    return pl.BlockSpec(shape, lambda i: tuple(0 for _ in shape))


def _run(body, grid, in_arrays, in_specs, out_shapes, out_specs):
    return pl.pallas_call(
        body,
        grid=(grid,),
        in_specs=in_specs,
        out_specs=out_specs,
        out_shape=out_shapes,
    )(*in_arrays)


def kernel(x, edge_index, batch, params):
    n = x.shape[0]
    in_dim = x.shape[1]
    h_dim = params['layers'][0]['W1'].shape[1]
    n_graphs = params['Wq1'].shape[0] - h_dim
    bn = 2000
    assert n % bn == 0
    grid = n // bn

    src = edge_index[0]
    dst = edge_index[1]
    bat2d = batch.reshape(n, 1).astype(jnp.int32)

    layers = params['layers']

    z = _run(_p0_body, grid,
             [x, layers[0]['W1']],
             [_row_spec(bn, in_dim), _const_spec((in_dim, h_dim))],
             jax.ShapeDtypeStruct((n, h_dim), jnp.float32),
             _row_spec(bn, h_dim))

    nf = float(n)
    for li, lp in enumerate(layers):
        agg = jnp.zeros_like(z).at[dst].add(z[src])

        e1p = (1.0 + lp['eps']).reshape(1, 1).astype(jnp.float32)
        b1 = lp['b1'].reshape(1, h_dim)
        u, st = _run(
            _p1_body, grid,
            [z, agg, e1p, b1],
            [_row_spec(bn, h_dim), _row_spec(bn, h_dim),
             _const_spec((1, 1)), _const_spec((1, h_dim))],
            (jax.ShapeDtypeStruct((n, h_dim), jnp.float32),
             jax.ShapeDtypeStruct((2, h_dim), jnp.float32)),
            (_row_spec(bn, h_dim), _const_spec((2, h_dim))))

        v, st2 = _run(
            functools.partial(_p2_body, n_nodes=nf), grid,
            [u, st, lp['bn1_g'].reshape(1, h_dim), lp['bn1_b'].reshape(1, h_dim),
             lp['W2'], lp['b2'].reshape(1, h_dim)],
            [_row_spec(bn, h_dim), _const_spec((2, h_dim)),
             _const_spec((1, h_dim)), _const_spec((1, h_dim)),
             _const_spec((h_dim, h_dim)), _const_spec((1, h_dim))],
            (jax.ShapeDtypeStruct((n, h_dim), jnp.float32),
             jax.ShapeDtypeStruct((2, h_dim), jnp.float32)),
            (_row_spec(bn, h_dim), _const_spec((2, h_dim))))

        g2 = lp['obn_g'].reshape(1, h_dim)
        bb2 = lp['obn_b'].reshape(1, h_dim)
        if li + 1 < len(layers):
            z = _run(
                functools.partial(_p3_body, n_nodes=nf), grid,
                [v, st2, g2, bb2, layers[li + 1]['W1']],
                [_row_spec(bn, h_dim), _const_spec((2, h_dim)),
                 _const_spec((1, h_dim)), _const_spec((1, h_dim)),
                 _const_spec((h_dim, h_dim))],
                jax.ShapeDtypeStruct((n, h_dim), jnp.float32),
                _row_spec(bn, h_dim))
        else:
            wq1b = params['Wq1'][h_dim:]
            h, ge2, _pool, _cnt = _run(
                functools.partial(_p3f_body, n_nodes=nf, n_graphs=n_graphs),
                grid,
                [v, st2, g2, bb2, bat2d, wq1b],
                [_row_spec(bn, h_dim), _const_spec((2, h_dim)),
                 _const_spec((1, h_dim)), _const_spec((1, h_dim)),
                 _row_spec(bn, 1), _const_spec((h_dim, h_dim))],
                (jax.ShapeDtypeStruct((n, h_dim), jnp.float32),
                 jax.ShapeDtypeStruct((n_graphs, h_dim), jnp.float32),
                 jax.ShapeDtypeStruct((n_graphs, h_dim), jnp.float32),
                 jax.ShapeDtypeStruct((n_graphs, 1), jnp.float32)),
                (_row_spec(bn, h_dim), _const_spec((n_graphs, h_dim)),
                 _const_spec((n_graphs, h_dim)), _const_spec((n_graphs, 1))))

    wq1t = params['Wq1'][:h_dim]
    wq2 = params['Wq2'].reshape(1, h_dim)
    bq1 = params['bq1'].reshape(1, h_dim)
    bq2 = params['bq2'].reshape(1, 1)
    q = _run(
        functools.partial(_p4_body, n_graphs=n_graphs), grid,
        [h, bat2d, ge2, wq1t, bq1, wq2, bq2],
        [_row_spec(bn, h_dim), _row_spec(bn, 1), _const_spec((n_graphs, h_dim)),
         _const_spec((h_dim, h_dim)), _const_spec((1, h_dim)),
         _const_spec((1, h_dim)), _const_spec((1, 1))],
        jax.ShapeDtypeStruct((n, 1), jnp.float32),
        _row_spec(bn, 1))
    return q.reshape(n)


# speedup vs baseline: 6.6434x; 6.4692x over previous
import functools

import jax
import jax.numpy as jnp
from jax import lax
from jax.experimental import pallas as pl
from jax.experimental.pallas import tpu as pltpu
from jax.experimental.pallas import tpu_sc as plsc


def _dot(a, b):
    return jnp.dot(a, b, preferred_element_type=jnp.float32,
                   precision=jax.lax.Precision.DEFAULT)


def _dot_t(a, b):
    return jax.lax.dot_general(a, b, (((0,), (0,)), ((), ())),
                               preferred_element_type=jnp.float32,
                               precision=jax.lax.Precision.DEFAULT)


_SC_TILES = 16
_SC_G = 16


def _sc_loop(table, srcb, dstb, rowsb, acc, gsem, ssem,
             srcr_hbm, dstr_hbm, base_row, rounds, g):
    def round_body(r, carry):
        r0 = base_row + r * g
        pltpu.sync_copy(srcr_hbm.at[pl.ds(r0, g)], srcb)
        pltpu.sync_copy(dstr_hbm.at[pl.ds(r0, g)], dstb)
        gathers = [
            pltpu.async_copy(table.at[srcb.at[i]], rowsb.at[i], gsem)
            for i in range(g)
        ]
        for gd in gathers:
            gd.wait()
        scatters = [
            pltpu.async_copy(rowsb.at[i], acc.at[dstb.at[i]], ssem, add=True)
            for i in range(g)
        ]
        for sd in scatters:
            sd.wait()
        return carry

    lax.fori_loop(0, rounds, round_body, 0)


def _make_sc_scatter4(n, hq, r_pad, acc_n):
    rows_per_tile = r_pad // _SC_TILES
    rounds = rows_per_tile // _SC_G
    zrows = acc_n // _SC_TILES
    wb = 2000
    n_wb = n // wb
    mesh = plsc.VectorSubcoreMesh(core_axis_name="c", subcore_axis_name="s")

    @functools.partial(
        pl.kernel,
        out_type=jax.ShapeDtypeStruct((4, n, hq), jnp.float32),
        mesh=mesh,
        scratch_types=[
            pltpu.VMEM((_SC_G, 128), jnp.int32),
            pltpu.VMEM((_SC_G, 128), jnp.int32),
            pltpu.VMEM((_SC_G, 128, hq), jnp.float32),
            pltpu.VMEM_SHARED((acc_n, hq), jnp.float32),
            pltpu.SemaphoreType.DMA,
            pltpu.SemaphoreType.DMA,
        ],
        compiler_params=pltpu.CompilerParams(use_tc_tiling_on_sc=False),
    )
    def sc_scatter(h_hbm, srcr_hbm, dstr_hbm, zeros_hbm, agg_hbm,
                   srcb, dstb, rowsb, acc, gsem, ssem):
        c = lax.axis_index("c")
        s = lax.axis_index("s")
        base_row = s * rows_per_tile

        for phase in range(2):
            t = 2 * phase + c

            pltpu.sync_copy(zeros_hbm, acc.at[pl.ds(s * zrows, zrows)])
            plsc.subcore_barrier()

            _sc_loop(h_hbm.at[t], srcb, dstb, rowsb, acc, gsem, ssem,
                     srcr_hbm, dstr_hbm, base_row, rounds, _SC_G)
            plsc.subcore_barrier()

            def _wb(ck, t=t):
                @pl.when(ck < n_wb)
                def _():
                    pltpu.sync_copy(acc.at[pl.ds(ck * wb, wb)],
                                    agg_hbm.at[t].at[pl.ds(ck * wb, wb)])

            for k in range(-(-n_wb // _SC_TILES)):
                _wb(s + _SC_TILES * k)
            plsc.subcore_barrier()

    return sc_scatter


def _make_sc_scatter1(n, w8, r_pad, acc_n):
    g = 8
    rows_per_tile = r_pad // (2 * _SC_TILES)
    rounds = rows_per_tile // g
    zrows = acc_n // _SC_TILES
    wb = 2000
    n_wb = n // wb
    mesh = plsc.VectorSubcoreMesh(core_axis_name="c", subcore_axis_name="s")

    @functools.partial(
        pl.kernel,
        out_type=jax.ShapeDtypeStruct((2, n, w8), jnp.float32),
        mesh=mesh,
        scratch_types=[
            pltpu.VMEM((g, 128), jnp.int32),
            pltpu.VMEM((g, 128), jnp.int32),
            pltpu.VMEM((g, 128, w8), jnp.float32),
            pltpu.VMEM_SHARED((acc_n, w8), jnp.float32),
            pltpu.SemaphoreType.DMA,
            pltpu.SemaphoreType.DMA,
        ],
        compiler_params=pltpu.CompilerParams(use_tc_tiling_on_sc=False),
    )
    def sc_scatter(x_hbm, srcr_hbm, dstr_hbm, zeros_hbm, agg_hbm,
                   srcb, dstb, rowsb, acc, gsem, ssem):
        c = lax.axis_index("c")
        s = lax.axis_index("s")
        base_row = (c * _SC_TILES + s) * rows_per_tile

        pltpu.sync_copy(zeros_hbm, acc.at[pl.ds(s * zrows, zrows)])
        plsc.subcore_barrier()

        _sc_loop(x_hbm, srcb, dstb, rowsb, acc, gsem, ssem,
                 srcr_hbm, dstr_hbm, base_row, rounds, g)
        plsc.subcore_barrier()

        def _wb(ck):
            @pl.when(ck < n_wb)
            def _():
                pltpu.sync_copy(acc.at[pl.ds(ck * wb, wb)],
                                agg_hbm.at[c].at[pl.ds(ck * wb, wb)])

        for k in range(-(-n_wb // _SC_TILES)):
            _wb(s + _SC_TILES * k)

    return sc_scatter


def _stats_acc(i, val, st_ref):
    @pl.when(i == 0)
    def _():
        st_ref[...] = jnp.zeros_like(st_ref)

    st_ref[0:1, :] += jnp.sum(val, axis=0, keepdims=True)
    st_ref[1:2, :] += jnp.sum(val * val, axis=0, keepdims=True)


def _p1a_body(x_ref, agg_ref, e1p_ref, w1_ref, b1_ref, u_ref, st_ref):
    pre = (e1p_ref[...] * x_ref[...] + agg_ref[0, ...] + agg_ref[1, ...])
    u = _dot(pre, w1_ref[...]) + b1_ref[...]
    u_ref[...] = u
    _stats_acc(pl.program_id(0), u, st_ref)


def _p1_body(h_ref, agg_ref, e1p_ref, w1_ref, b1_ref, u_ref, st_ref):
    e1p = e1p_ref[...]
    pre = jnp.concatenate(
        [e1p * h_ref[t, ...] + agg_ref[t, ...]
         for t in range(h_ref.shape[0])], axis=1)
    u = _dot(pre, w1_ref[...]) + b1_ref[...]
    u_ref[...] = u
    _stats_acc(pl.program_id(0), u, st_ref)


def _bn(val, st, g, b, n):
    mean = st[0:1, :] / n
    var = st[1:2, :] / n - mean * mean
    return g * (val - mean) / jnp.sqrt(var + 1e-5) + b


def _p2_body(u_ref, st_ref, g_ref, b_ref, w2_ref, b2_ref, v_ref, st2_ref, *,
             n_nodes):
    r = jnp.maximum(_bn(u_ref[...], st_ref[...], g_ref[...], b_ref[...],
                        n_nodes), 0.0)
    v = _dot(r, w2_ref[...]) + b2_ref[...]
    v_ref[...] = v
    _stats_acc(pl.program_id(0), v, st2_ref)


def _p3_body(v_ref, st_ref, g_ref, b_ref, hn_ref, *, n_nodes, hq):
    h = jnp.maximum(_bn(v_ref[...], st_ref[...], g_ref[...], b_ref[...],
                        n_nodes), 0.0)
    for t in range(hn_ref.shape[0]):
        hn_ref[t, ...] = h[:, t * hq:(t + 1) * hq]


def _p3f_body(v_ref, st_ref, g_ref, b_ref, bat_ref,
              h_ref, ge_ref, pool_ref, cnt_ref, *, n_nodes, n_graphs):
    i = pl.program_id(0)
    h = jnp.maximum(_bn(v_ref[...], st_ref[...], g_ref[...], b_ref[...],
                        n_nodes), 0.0)
    h_ref[...] = h

    bids = bat_ref[...]
    lanes = jax.lax.broadcasted_iota(jnp.int32, (bids.shape[0], n_graphs), 1)
    onehot = (bids == lanes).astype(jnp.float32)

    @pl.when(i == 0)
    def _():
        pool_ref[...] = jnp.zeros_like(pool_ref)
        cnt_ref[...] = jnp.zeros_like(cnt_ref)

    pool_ref[...] += _dot_t(onehot, h)
    cnt_ref[...] += _dot_t(onehot, jnp.ones((bids.shape[0], 1), jnp.float32))

    @pl.when(i == pl.num_programs(0) - 1)
    def _():
        cnt = jnp.maximum(cnt_ref[...], 1.0)
        ge_ref[...] = pool_ref[...] / cnt


def _p4_body(h_ref, bat_ref, ge_ref, wq1_ref, bq1_ref, wq2_ref, bq2_ref,
             q_ref, *, n_graphs):
    bids = bat_ref[...]
    lanes = jax.lax.broadcasted_iota(jnp.int32, (bids.shape[0], n_graphs), 1)
    onehot = (bids == lanes).astype(jnp.float32)
    gepn = _dot(onehot, ge_ref[...])
    cc = jnp.concatenate([h_ref[...], gepn], axis=1)
    r = jnp.maximum(_dot(cc, wq1_ref[...]) + bq1_ref[...], 0.0)
    q = jnp.sum(r * wq2_ref[...], axis=1, keepdims=True) + bq2_ref[...]
    q_ref[...] = q


def _row_spec(bn, w):
    return pl.BlockSpec((bn, w), lambda i: (i, 0))


def _grp_spec(ng, bn, w):
    return pl.BlockSpec((ng, bn, w), lambda i: (0, i, 0))


def _const_spec(shape):
    return pl.BlockSpec(shape, lambda i: tuple(0 for _ in shape))


def _run(body, grid, in_arrays, in_specs, out_shapes, out_specs):
    return pl.pallas_call(
        body,
        grid=(grid,),
        in_specs=in_specs,
        out_specs=out_specs,
        out_shape=out_shapes,
    )(*in_arrays)


def kernel(x, edge_index, batch, params):
    n = x.shape[0]
    in_dim = x.shape[1]
    w8 = 8
    h_dim = params['layers'][0]['W1'].shape[1]
    hq = h_dim // 4
    n_graphs = params['Wq1'].shape[0] - h_dim
    bn = 2000
    assert n % bn == 0
    grid = n // bn

    e = edge_index.shape[1]
    r_pad = -((-e) // (128 * _SC_TILES * _SC_G)) * (_SC_TILES * _SC_G)
    e_pad = r_pad * 128 - e
    acc_n = -((-(n + 128)) // (_SC_TILES * 8)) * (_SC_TILES * 8)

    src = edge_index[0].astype(jnp.int32)
    dst = edge_index[1].astype(jnp.int32)
    pad_ids = jnp.arange(e_pad, dtype=jnp.int32)
    srcr = jnp.concatenate([src, pad_ids % n]).reshape(r_pad, 128)
    dstr = jnp.concatenate([dst, n + (pad_ids % 128)]).reshape(r_pad, 128)
    zeros1 = jnp.zeros((acc_n // _SC_TILES, w8), jnp.float32)
    zeros4 = jnp.zeros((acc_n // _SC_TILES, hq), jnp.float32)
    sc_scatter1 = _make_sc_scatter1(n, w8, r_pad, acc_n)
    sc_scatter4 = _make_sc_scatter4(n, hq, r_pad, acc_n)

    x_pad = jnp.pad(x, ((0, 0), (0, w8 - in_dim)))
    bat2d = batch.reshape(n, 1).astype(jnp.int32)
    layers = params['layers']
    nf = float(n)

    h = None
    for li, lp in enumerate(layers):
        e1p = (1.0 + lp['eps']).reshape(1, 1).astype(jnp.float32)
        b1 = lp['b1'].reshape(1, h_dim)
        if li == 0:
            agg = sc_scatter1(x_pad, srcr, dstr, zeros1)
            w1 = jnp.pad(lp['W1'], ((0, w8 - in_dim), (0, 0)))
            u, st = _run(
                _p1a_body, grid,
                [x_pad, agg, e1p, w1, b1],
                [_row_spec(bn, w8), _grp_spec(2, bn, w8),
                 _const_spec((1, 1)), _const_spec((w8, h_dim)),
                 _const_spec((1, h_dim))],
                (jax.ShapeDtypeStruct((n, h_dim), jnp.float32),
                 jax.ShapeDtypeStruct((2, h_dim), jnp.float32)),
                (_row_spec(bn, h_dim), _const_spec((2, h_dim))))
        else:
            agg = sc_scatter4(h, srcr, dstr, zeros4)
            u, st = _run(
                _p1_body, grid,
                [h, agg, e1p, lp['W1'], b1],
                [_grp_spec(4, bn, hq), _grp_spec(4, bn, hq),
                 _const_spec((1, 1)), _const_spec((h_dim, h_dim)),
                 _const_spec((1, h_dim))],
                (jax.ShapeDtypeStruct((n, h_dim), jnp.float32),
                 jax.ShapeDtypeStruct((2, h_dim), jnp.float32)),
                (_row_spec(bn, h_dim), _const_spec((2, h_dim))))

        v, st2 = _run(
            functools.partial(_p2_body, n_nodes=nf), grid,
            [u, st, lp['bn1_g'].reshape(1, h_dim), lp['bn1_b'].reshape(1, h_dim),
             lp['W2'], lp['b2'].reshape(1, h_dim)],
            [_row_spec(bn, h_dim), _const_spec((2, h_dim)),
             _const_spec((1, h_dim)), _const_spec((1, h_dim)),
             _const_spec((h_dim, h_dim)), _const_spec((1, h_dim))],
            (jax.ShapeDtypeStruct((n, h_dim), jnp.float32),
             jax.ShapeDtypeStruct((2, h_dim), jnp.float32)),
            (_row_spec(bn, h_dim), _const_spec((2, h_dim))))

        g2 = lp['obn_g'].reshape(1, h_dim)
        bb2 = lp['obn_b'].reshape(1, h_dim)
        if li + 1 < len(layers):
            h = _run(
                functools.partial(_p3_body, n_nodes=nf, hq=hq), grid,
                [v, st2, g2, bb2],
                [_row_spec(bn, h_dim), _const_spec((2, h_dim)),
                 _const_spec((1, h_dim)), _const_spec((1, h_dim))],
                jax.ShapeDtypeStruct((4, n, hq), jnp.float32),
                _grp_spec(4, bn, hq))
        else:
            hf, ge, _pool, _cnt = _run(
                functools.partial(_p3f_body, n_nodes=nf, n_graphs=n_graphs),
                grid,
                [v, st2, g2, bb2, bat2d],
                [_row_spec(bn, h_dim), _const_spec((2, h_dim)),
                 _const_spec((1, h_dim)), _const_spec((1, h_dim)),
                 _row_spec(bn, 1)],
                (jax.ShapeDtypeStruct((n, h_dim), jnp.float32),
                 jax.ShapeDtypeStruct((n_graphs, h_dim), jnp.float32),
                 jax.ShapeDtypeStruct((n_graphs, h_dim), jnp.float32),
                 jax.ShapeDtypeStruct((n_graphs, 1), jnp.float32)),
                (_row_spec(bn, h_dim), _const_spec((n_graphs, h_dim)),
                 _const_spec((n_graphs, h_dim)), _const_spec((n_graphs, 1))))

    wq2 = params['Wq2'].reshape(1, h_dim)
    bq1 = params['bq1'].reshape(1, h_dim)
    bq2 = params['bq2'].reshape(1, 1)
    q = _run(
        functools.partial(_p4_body, n_graphs=n_graphs), grid,
        [hf, bat2d, ge, params['Wq1'], bq1, wq2, bq2],
        [_row_spec(bn, h_dim), _row_spec(bn, 1), _const_spec((n_graphs, h_dim)),
         _const_spec((2 * h_dim, h_dim)), _const_spec((1, h_dim)),
         _const_spec((1, h_dim)), _const_spec((1, 1))],
        jax.ShapeDtypeStruct((n, 1), jnp.float32),
        _row_spec(bn, 1))
    return q.reshape(n)


# speedup vs baseline: 7.1675x; 1.0789x over previous
import functools

import jax
import jax.numpy as jnp
from jax import lax
from jax.experimental import pallas as pl
from jax.experimental.pallas import tpu as pltpu
from jax.experimental.pallas import tpu_sc as plsc


def _dot(a, b):
    return jnp.dot(a, b, preferred_element_type=jnp.float32,
                   precision=jax.lax.Precision.DEFAULT)


def _dot_t(a, b):
    return jax.lax.dot_general(a, b, (((0,), (0,)), ((), ())),
                               preferred_element_type=jnp.float32,
                               precision=jax.lax.Precision.DEFAULT)


_SC_TILES = 16
_SC_G = 16


def _sc_loop(table, srcb, dstb, rowsb, acc, gsems, ssems,
             srcr_hbm, dstr_hbm, base_row, rounds, g):
    hg = g // 2

    def _load_idx(h, r0):
        pltpu.sync_copy(srcr_hbm.at[pl.ds(r0, hg)], srcb.at[h])
        pltpu.sync_copy(dstr_hbm.at[pl.ds(r0, hg)], dstb.at[h])

    def _fire_gathers(h):
        return [pltpu.async_copy(table.at[srcb.at[h, i]], rowsb.at[h, i],
                                 gsems[h]) for i in range(hg)]

    def _drain_gathers(h):
        for i in range(hg):
            pltpu.make_async_copy(table.at[srcb.at[h, i]], rowsb.at[h, i],
                                  gsems[h]).wait()

    def _fire_scatters(h):
        return [pltpu.async_copy(rowsb.at[h, i], acc.at[dstb.at[h, i]],
                                 ssems[h], add=True) for i in range(hg)]

    def _drain_scatters(h):
        for i in range(hg):
            pltpu.make_async_copy(rowsb.at[h, i], acc.at[dstb.at[h, i]],
                                  ssems[h]).wait()

    _load_idx(0, base_row)
    _fire_gathers(0)

    def round_body(r, carry):
        r0 = base_row + r * g
        _load_idx(1, r0 + hg)
        _fire_gathers(1)
        _drain_gathers(0)
        _fire_scatters(0)
        _drain_gathers(1)
        _fire_scatters(1)
        _drain_scatters(0)

        @pl.when(r + 1 < rounds)
        def _():
            _load_idx(0, r0 + g)
            _fire_gathers(0)

        _drain_scatters(1)
        return carry

    lax.fori_loop(0, rounds, round_body, 0)


def _make_sc_scatter4(n, hq, r_pad, acc_n):
    rows_per_tile = r_pad // _SC_TILES
    rounds = rows_per_tile // _SC_G
    zrows = acc_n // _SC_TILES
    wb = 2000
    n_wb = n // wb
    mesh = plsc.VectorSubcoreMesh(core_axis_name="c", subcore_axis_name="s")

    @functools.partial(
        pl.kernel,
        out_type=jax.ShapeDtypeStruct((4, n, hq), jnp.float32),
        mesh=mesh,
        scratch_types=[
            pltpu.VMEM((2, _SC_G // 2, 128), jnp.int32),
            pltpu.VMEM((2, _SC_G // 2, 128), jnp.int32),
            pltpu.VMEM((2, _SC_G // 2, 128, hq), jnp.float32),
            pltpu.VMEM_SHARED((acc_n, hq), jnp.float32),
            pltpu.SemaphoreType.DMA,
            pltpu.SemaphoreType.DMA,
            pltpu.SemaphoreType.DMA,
            pltpu.SemaphoreType.DMA,
        ],
        compiler_params=pltpu.CompilerParams(use_tc_tiling_on_sc=False),
    )
    def sc_scatter(h_hbm, srcr_hbm, dstr_hbm, zeros_hbm, agg_hbm,
                   srcb, dstb, rowsb, acc, gsem0, gsem1, ssem0, ssem1):
        c = lax.axis_index("c")
        s = lax.axis_index("s")
        base_row = s * rows_per_tile

        for phase in range(2):
            t = 2 * phase + c

            pltpu.sync_copy(zeros_hbm, acc.at[pl.ds(s * zrows, zrows)])
            plsc.subcore_barrier()

            _sc_loop(h_hbm.at[t], srcb, dstb, rowsb, acc,
                     (gsem0, gsem1), (ssem0, ssem1),
                     srcr_hbm, dstr_hbm, base_row, rounds, _SC_G)
            plsc.subcore_barrier()

            def _wb(ck, t=t):
                @pl.when(ck < n_wb)
                def _():
                    pltpu.sync_copy(acc.at[pl.ds(ck * wb, wb)],
                                    agg_hbm.at[t].at[pl.ds(ck * wb, wb)])

            for k in range(-(-n_wb // _SC_TILES)):
                _wb(s + _SC_TILES * k)
            plsc.subcore_barrier()

    return sc_scatter


def _make_sc_scatter1(n, w8, r_pad, acc_n):
    g = 8
    rows_per_tile = r_pad // (2 * _SC_TILES)
    rounds = rows_per_tile // g
    zrows = acc_n // _SC_TILES
    wb = 2000
    n_wb = n // wb
    mesh = plsc.VectorSubcoreMesh(core_axis_name="c", subcore_axis_name="s")

    @functools.partial(
        pl.kernel,
        out_type=jax.ShapeDtypeStruct((2, n, w8), jnp.float32),
        mesh=mesh,
        scratch_types=[
            pltpu.VMEM((2, g // 2, 128), jnp.int32),
            pltpu.VMEM((2, g // 2, 128), jnp.int32),
            pltpu.VMEM((2, g // 2, 128, w8), jnp.float32),
            pltpu.VMEM_SHARED((acc_n, w8), jnp.float32),
            pltpu.SemaphoreType.DMA,
            pltpu.SemaphoreType.DMA,
            pltpu.SemaphoreType.DMA,
            pltpu.SemaphoreType.DMA,
        ],
        compiler_params=pltpu.CompilerParams(use_tc_tiling_on_sc=False),
    )
    def sc_scatter(x_hbm, srcr_hbm, dstr_hbm, zeros_hbm, agg_hbm,
                   srcb, dstb, rowsb, acc, gsem0, gsem1, ssem0, ssem1):
        c = lax.axis_index("c")
        s = lax.axis_index("s")
        base_row = (c * _SC_TILES + s) * rows_per_tile

        pltpu.sync_copy(zeros_hbm, acc.at[pl.ds(s * zrows, zrows)])
        plsc.subcore_barrier()

        _sc_loop(x_hbm, srcb, dstb, rowsb, acc,
                 (gsem0, gsem1), (ssem0, ssem1),
                 srcr_hbm, dstr_hbm, base_row, rounds, g)
        plsc.subcore_barrier()

        def _wb(ck):
            @pl.when(ck < n_wb)
            def _():
                pltpu.sync_copy(acc.at[pl.ds(ck * wb, wb)],
                                agg_hbm.at[c].at[pl.ds(ck * wb, wb)])

        for k in range(-(-n_wb // _SC_TILES)):
            _wb(s + _SC_TILES * k)

    return sc_scatter


def _stats_acc(i, val, st_ref):
    @pl.when(i == 0)
    def _():
        st_ref[...] = jnp.zeros_like(st_ref)

    st_ref[0:1, :] += jnp.sum(val, axis=0, keepdims=True)
    st_ref[1:2, :] += jnp.sum(val * val, axis=0, keepdims=True)


def _p1a_body(x_ref, agg_ref, e1p_ref, w1_ref, b1_ref, u_ref, st_ref):
    pre = (e1p_ref[...] * x_ref[...] + agg_ref[0, ...] + agg_ref[1, ...])
    u = _dot(pre, w1_ref[...]) + b1_ref[...]
    u_ref[...] = u
    _stats_acc(pl.program_id(0), u, st_ref)


def _p1_body(h_ref, agg_ref, e1p_ref, w1_ref, b1_ref, u_ref, st_ref):
    e1p = e1p_ref[...]
    pre = jnp.concatenate(
        [e1p * h_ref[t, ...] + agg_ref[t, ...]
         for t in range(h_ref.shape[0])], axis=1)
    u = _dot(pre, w1_ref[...]) + b1_ref[...]
    u_ref[...] = u
    _stats_acc(pl.program_id(0), u, st_ref)


def _bn(val, st, g, b, n):
    mean = st[0:1, :] / n
    var = st[1:2, :] / n - mean * mean
    return g * (val - mean) / jnp.sqrt(var + 1e-5) + b


def _p2_body(u_ref, st_ref, g_ref, b_ref, w2_ref, b2_ref, v_ref, st2_ref, *,
             n_nodes):
    r = jnp.maximum(_bn(u_ref[...], st_ref[...], g_ref[...], b_ref[...],
                        n_nodes), 0.0)
    v = _dot(r, w2_ref[...]) + b2_ref[...]
    v_ref[...] = v
    _stats_acc(pl.program_id(0), v, st2_ref)


def _p3_body(v_ref, st_ref, g_ref, b_ref, hn_ref, *, n_nodes, hq):
    h = jnp.maximum(_bn(v_ref[...], st_ref[...], g_ref[...], b_ref[...],
                        n_nodes), 0.0)
    for t in range(hn_ref.shape[0]):
        hn_ref[t, ...] = h[:, t * hq:(t + 1) * hq]


def _p3f_body(v_ref, st_ref, g_ref, b_ref, bat_ref,
              h_ref, ge_ref, pool_ref, cnt_ref, *, n_nodes, n_graphs):
    i = pl.program_id(0)
    h = jnp.maximum(_bn(v_ref[...], st_ref[...], g_ref[...], b_ref[...],
                        n_nodes), 0.0)
    h_ref[...] = h

    bids = bat_ref[...]
    lanes = jax.lax.broadcasted_iota(jnp.int32, (bids.shape[0], n_graphs), 1)
    onehot = (bids == lanes).astype(jnp.float32)

    @pl.when(i == 0)
    def _():
        pool_ref[...] = jnp.zeros_like(pool_ref)
        cnt_ref[...] = jnp.zeros_like(cnt_ref)

    pool_ref[...] += _dot_t(onehot, h)
    cnt_ref[...] += _dot_t(onehot, jnp.ones((bids.shape[0], 1), jnp.float32))

    @pl.when(i == pl.num_programs(0) - 1)
    def _():
        cnt = jnp.maximum(cnt_ref[...], 1.0)
        ge_ref[...] = pool_ref[...] / cnt


def _p4_body(h_ref, bat_ref, ge_ref, wq1_ref, bq1_ref, wq2_ref, bq2_ref,
             q_ref, *, n_graphs):
    bids = bat_ref[...]
    lanes = jax.lax.broadcasted_iota(jnp.int32, (bids.shape[0], n_graphs), 1)
    onehot = (bids == lanes).astype(jnp.float32)
    gepn = _dot(onehot, ge_ref[...])
    cc = jnp.concatenate([h_ref[...], gepn], axis=1)
    r = jnp.maximum(_dot(cc, wq1_ref[...]) + bq1_ref[...], 0.0)
    q = jnp.sum(r * wq2_ref[...], axis=1, keepdims=True) + bq2_ref[...]
    q_ref[...] = q


def _row_spec(bn, w):
    return pl.BlockSpec((bn, w), lambda i: (i, 0))


def _grp_spec(ng, bn, w):
    return pl.BlockSpec((ng, bn, w), lambda i: (0, i, 0))


def _const_spec(shape):
    return pl.BlockSpec(shape, lambda i: tuple(0 for _ in shape))


def _run(body, grid, in_arrays, in_specs, out_shapes, out_specs):
    return pl.pallas_call(
        body,
        grid=(grid,),
        in_specs=in_specs,
        out_specs=out_specs,
        out_shape=out_shapes,
    )(*in_arrays)


def kernel(x, edge_index, batch, params):
    n = x.shape[0]
    in_dim = x.shape[1]
    w8 = 8
    h_dim = params['layers'][0]['W1'].shape[1]
    hq = h_dim // 4
    n_graphs = params['Wq1'].shape[0] - h_dim
    bn = 2000
    assert n % bn == 0
    grid = n // bn

    e = edge_index.shape[1]
    r_pad = -((-e) // (128 * _SC_TILES * _SC_G)) * (_SC_TILES * _SC_G)
    e_pad = r_pad * 128 - e
    acc_n = -((-(n + 128)) // (_SC_TILES * 8)) * (_SC_TILES * 8)

    src = edge_index[0].astype(jnp.int32)
    dst = edge_index[1].astype(jnp.int32)
    pad_ids = jnp.arange(e_pad, dtype=jnp.int32)
    srcr = jnp.concatenate([src, pad_ids % n]).reshape(r_pad, 128)
    dstr = jnp.concatenate([dst, n + (pad_ids % 128)]).reshape(r_pad, 128)
    zeros1 = jnp.zeros((acc_n // _SC_TILES, w8), jnp.float32)
    zeros4 = jnp.zeros((acc_n // _SC_TILES, hq), jnp.float32)
    sc_scatter1 = _make_sc_scatter1(n, w8, r_pad, acc_n)
    sc_scatter4 = _make_sc_scatter4(n, hq, r_pad, acc_n)

    x_pad = jnp.pad(x, ((0, 0), (0, w8 - in_dim)))
    bat2d = batch.reshape(n, 1).astype(jnp.int32)
    layers = params['layers']
    nf = float(n)

    h = None
    for li, lp in enumerate(layers):
        e1p = (1.0 + lp['eps']).reshape(1, 1).astype(jnp.float32)
        b1 = lp['b1'].reshape(1, h_dim)
        if li == 0:
            agg = sc_scatter1(x_pad, srcr, dstr, zeros1)
            w1 = jnp.pad(lp['W1'], ((0, w8 - in_dim), (0, 0)))
            u, st = _run(
                _p1a_body, grid,
                [x_pad, agg, e1p, w1, b1],
                [_row_spec(bn, w8), _grp_spec(2, bn, w8),
                 _const_spec((1, 1)), _const_spec((w8, h_dim)),
                 _const_spec((1, h_dim))],
                (jax.ShapeDtypeStruct((n, h_dim), jnp.float32),
                 jax.ShapeDtypeStruct((2, h_dim), jnp.float32)),
                (_row_spec(bn, h_dim), _const_spec((2, h_dim))))
        else:
            agg = sc_scatter4(h, srcr, dstr, zeros4)
            u, st = _run(
                _p1_body, grid,
                [h, agg, e1p, lp['W1'], b1],
                [_grp_spec(4, bn, hq), _grp_spec(4, bn, hq),
                 _const_spec((1, 1)), _const_spec((h_dim, h_dim)),
                 _const_spec((1, h_dim))],
                (jax.ShapeDtypeStruct((n, h_dim), jnp.float32),
                 jax.ShapeDtypeStruct((2, h_dim), jnp.float32)),
                (_row_spec(bn, h_dim), _const_spec((2, h_dim))))

        v, st2 = _run(
            functools.partial(_p2_body, n_nodes=nf), grid,
            [u, st, lp['bn1_g'].reshape(1, h_dim), lp['bn1_b'].reshape(1, h_dim),
             lp['W2'], lp['b2'].reshape(1, h_dim)],
            [_row_spec(bn, h_dim), _const_spec((2, h_dim)),
             _const_spec((1, h_dim)), _const_spec((1, h_dim)),
             _const_spec((h_dim, h_dim)), _const_spec((1, h_dim))],
            (jax.ShapeDtypeStruct((n, h_dim), jnp.float32),
             jax.ShapeDtypeStruct((2, h_dim), jnp.float32)),
            (_row_spec(bn, h_dim), _const_spec((2, h_dim))))

        g2 = lp['obn_g'].reshape(1, h_dim)
        bb2 = lp['obn_b'].reshape(1, h_dim)
        if li + 1 < len(layers):
            h = _run(
                functools.partial(_p3_body, n_nodes=nf, hq=hq), grid,
                [v, st2, g2, bb2],
                [_row_spec(bn, h_dim), _const_spec((2, h_dim)),
                 _const_spec((1, h_dim)), _const_spec((1, h_dim))],
                jax.ShapeDtypeStruct((4, n, hq), jnp.float32),
                _grp_spec(4, bn, hq))
        else:
            hf, ge, _pool, _cnt = _run(
                functools.partial(_p3f_body, n_nodes=nf, n_graphs=n_graphs),
                grid,
                [v, st2, g2, bb2, bat2d],
                [_row_spec(bn, h_dim), _const_spec((2, h_dim)),
                 _const_spec((1, h_dim)), _const_spec((1, h_dim)),
                 _row_spec(bn, 1)],
                (jax.ShapeDtypeStruct((n, h_dim), jnp.float32),
                 jax.ShapeDtypeStruct((n_graphs, h_dim), jnp.float32),
                 jax.ShapeDtypeStruct((n_graphs, h_dim), jnp.float32),
                 jax.ShapeDtypeStruct((n_graphs, 1), jnp.float32)),
                (_row_spec(bn, h_dim), _const_spec((n_graphs, h_dim)),
                 _const_spec((n_graphs, h_dim)), _const_spec((n_graphs, 1))))

    wq2 = params['Wq2'].reshape(1, h_dim)
    bq1 = params['bq1'].reshape(1, h_dim)
    bq2 = params['bq2'].reshape(1, 1)
    q = _run(
        functools.partial(_p4_body, n_graphs=n_graphs), grid,
        [hf, bat2d, ge, params['Wq1'], bq1, wq2, bq2],
        [_row_spec(bn, h_dim), _row_spec(bn, 1), _const_spec((n_graphs, h_dim)),
         _const_spec((2 * h_dim, h_dim)), _const_spec((1, h_dim)),
         _const_spec((1, h_dim)), _const_spec((1, 1))],
        jax.ShapeDtypeStruct((n, 1), jnp.float32),
        _row_spec(bn, 1))
    return q.reshape(n)


# speedup vs baseline: 7.5494x; 1.0533x over previous
import functools

import jax
import jax.numpy as jnp
from jax import lax
from jax.experimental import pallas as pl
from jax.experimental.pallas import tpu as pltpu
from jax.experimental.pallas import tpu_sc as plsc


def _dot(a, b):
    return jnp.dot(a, b, preferred_element_type=jnp.float32,
                   precision=jax.lax.Precision.DEFAULT)


def _dot_t(a, b):
    return jax.lax.dot_general(a, b, (((0,), (0,)), ((), ())),
                               preferred_element_type=jnp.float32,
                               precision=jax.lax.Precision.DEFAULT)


_SC_TILES = 16
_SC_G = 16


def _sc_loop(table, srcb, dstb, rowsb, acc, gsems, ssems,
             srcr_hbm, dstr_hbm, base_row, rounds, g):
    hg = g // 2

    def _load_idx(h, r0):
        pltpu.sync_copy(srcr_hbm.at[pl.ds(r0, hg)], srcb.at[h])
        pltpu.sync_copy(dstr_hbm.at[pl.ds(r0, hg)], dstb.at[h])

    def _fire_gathers(h):
        return [pltpu.async_copy(table.at[srcb.at[h, i]], rowsb.at[h, i],
                                 gsems[h]) for i in range(hg)]

    def _drain_gathers(h):
        for i in range(hg):
            pltpu.make_async_copy(table.at[srcb.at[h, i]], rowsb.at[h, i],
                                  gsems[h]).wait()

    def _fire_scatters(h):
        return [pltpu.async_copy(rowsb.at[h, i], acc.at[dstb.at[h, i]],
                                 ssems[h], add=True) for i in range(hg)]

    def _drain_scatters(h):
        for i in range(hg):
            pltpu.make_async_copy(rowsb.at[h, i], acc.at[dstb.at[h, i]],
                                  ssems[h]).wait()

    _load_idx(0, base_row)
    _fire_gathers(0)

    def round_body(r, carry):
        r0 = base_row + r * g
        _load_idx(1, r0 + hg)
        _fire_gathers(1)
        _drain_gathers(0)
        _fire_scatters(0)
        _drain_gathers(1)
        _fire_scatters(1)
        _drain_scatters(0)

        @pl.when(r + 1 < rounds)
        def _():
            _load_idx(0, r0 + g)
            _fire_gathers(0)

        _drain_scatters(1)
        return carry

    lax.fori_loop(0, rounds, round_body, 0)


def _make_sc_scatter4(n, hq, r_pad, acc_n):
    rows_per_tile = r_pad // _SC_TILES
    rounds = rows_per_tile // _SC_G
    zrows = acc_n // _SC_TILES
    wb = 2000
    n_wb = n // wb
    mesh = plsc.VectorSubcoreMesh(core_axis_name="c", subcore_axis_name="s")

    @functools.partial(
        pl.kernel,
        out_type=jax.ShapeDtypeStruct((4, n, hq), jnp.float32),
        mesh=mesh,
        scratch_types=[
            pltpu.VMEM((2, _SC_G // 2, 128), jnp.int32),
            pltpu.VMEM((2, _SC_G // 2, 128), jnp.int32),
            pltpu.VMEM((2, _SC_G // 2, 128, hq), jnp.float32),
            pltpu.VMEM_SHARED((acc_n, hq), jnp.float32),
            pltpu.SemaphoreType.DMA,
            pltpu.SemaphoreType.DMA,
            pltpu.SemaphoreType.DMA,
            pltpu.SemaphoreType.DMA,
        ],
        compiler_params=pltpu.CompilerParams(use_tc_tiling_on_sc=False),
    )
    def sc_scatter(h_hbm, srcr_hbm, dstr_hbm, zeros_hbm, agg_hbm,
                   srcb, dstb, rowsb, acc, gsem0, gsem1, ssem0, ssem1):
        c = lax.axis_index("c")
        s = lax.axis_index("s")
        base_row = s * rows_per_tile

        for phase in range(2):
            t = 2 * phase + c

            pltpu.sync_copy(zeros_hbm, acc.at[pl.ds(s * zrows, zrows)])
            plsc.subcore_barrier()

            _sc_loop(h_hbm.at[t], srcb, dstb, rowsb, acc,
                     (gsem0, gsem1), (ssem0, ssem1),
                     srcr_hbm, dstr_hbm, base_row, rounds, _SC_G)
            plsc.subcore_barrier()

            def _wb(ck, t=t):
                @pl.when(ck < n_wb)
                def _():
                    pltpu.sync_copy(acc.at[pl.ds(ck * wb, wb)],
                                    agg_hbm.at[t].at[pl.ds(ck * wb, wb)])

            for k in range(-(-n_wb // _SC_TILES)):
                _wb(s + _SC_TILES * k)
            plsc.subcore_barrier()

    return sc_scatter


def _make_sc_scatter1(n, w8, r_pad, acc_n):
    g = 8
    rows_per_tile = r_pad // (2 * _SC_TILES)
    rounds = rows_per_tile // g
    zrows = acc_n // _SC_TILES
    wb = 2000
    n_wb = n // wb
    mesh = plsc.VectorSubcoreMesh(core_axis_name="c", subcore_axis_name="s")

    @functools.partial(
        pl.kernel,
        out_type=jax.ShapeDtypeStruct((2, n, w8), jnp.float32),
        mesh=mesh,
        scratch_types=[
            pltpu.VMEM((2, g // 2, 128), jnp.int32),
            pltpu.VMEM((2, g // 2, 128), jnp.int32),
            pltpu.VMEM((2, g // 2, 128, w8), jnp.float32),
            pltpu.VMEM_SHARED((acc_n, w8), jnp.float32),
            pltpu.SemaphoreType.DMA,
            pltpu.SemaphoreType.DMA,
            pltpu.SemaphoreType.DMA,
            pltpu.SemaphoreType.DMA,
        ],
        compiler_params=pltpu.CompilerParams(use_tc_tiling_on_sc=False),
    )
    def sc_scatter(x_hbm, srcr_hbm, dstr_hbm, zeros_hbm, agg_hbm,
                   srcb, dstb, rowsb, acc, gsem0, gsem1, ssem0, ssem1):
        c = lax.axis_index("c")
        s = lax.axis_index("s")
        base_row = (c * _SC_TILES + s) * rows_per_tile

        pltpu.sync_copy(zeros_hbm, acc.at[pl.ds(s * zrows, zrows)])
        plsc.subcore_barrier()

        _sc_loop(x_hbm, srcb, dstb, rowsb, acc,
                 (gsem0, gsem1), (ssem0, ssem1),
                 srcr_hbm, dstr_hbm, base_row, rounds, g)
        plsc.subcore_barrier()

        def _wb(ck):
            @pl.when(ck < n_wb)
            def _():
                pltpu.sync_copy(acc.at[pl.ds(ck * wb, wb)],
                                agg_hbm.at[c].at[pl.ds(ck * wb, wb)])

        for k in range(-(-n_wb // _SC_TILES)):
            _wb(s + _SC_TILES * k)

    return sc_scatter


def _stats_acc(i, val, st_ref):
    @pl.when(i == 0)
    def _():
        st_ref[...] = jnp.zeros_like(st_ref)

    st_ref[0:1, :] += jnp.sum(val, axis=0, keepdims=True)
    st_ref[1:2, :] += jnp.sum(val * val, axis=0, keepdims=True)


def _bn(val, st, g, b, n):
    mean = st[0:1, :] / n
    var = st[1:2, :] / n - mean * mean
    return g * (val - mean) / jnp.sqrt(var + 1e-5) + b


def _gin_phases01(p, i, bn, hin_ref, agg_ref, e1p_ref, w1_ref, b1_ref,
                  g1_ref, bb1_ref, w2_ref, b2_ref, s_s, st1, st2, *,
                  nf, first):
    @pl.when(p == 0)
    def _():
        e1p = e1p_ref[...]
        if first:
            pre = e1p * hin_ref[...] + agg_ref[0, ...] + agg_ref[1, ...]
        else:
            pre = jnp.concatenate(
                [e1p * hin_ref[t, ...] + agg_ref[t, ...]
                 for t in range(hin_ref.shape[0])], axis=1)
        u = _dot(pre, w1_ref[...]) + b1_ref[...]
        s_s[pl.ds(i * bn, bn), :] = u
        _stats_acc(i, u, st1)

    @pl.when(p == 1)
    def _():
        u = s_s[pl.ds(i * bn, bn), :]
        r = jnp.maximum(_bn(u, st1[...], g1_ref[...], bb1_ref[...], nf), 0.0)
        v = _dot(r, w2_ref[...]) + b2_ref[...]
        s_s[pl.ds(i * bn, bn), :] = v
        _stats_acc(i, v, st2)


def _layer_mid_body(hin_ref, agg_ref, e1p_ref, w1_ref, b1_ref, g1_ref,
                    bb1_ref, w2_ref, b2_ref, g2_ref, bb2_ref, hn_ref,
                    s_s, st1, st2, *, nf, hq, bn, first):
    p = pl.program_id(0)
    i = pl.program_id(1)
    _gin_phases01(p, i, bn, hin_ref, agg_ref, e1p_ref, w1_ref, b1_ref,
                  g1_ref, bb1_ref, w2_ref, b2_ref, s_s, st1, st2,
                  nf=nf, first=first)

    @pl.when(p == 2)
    def _():
        v = s_s[pl.ds(i * bn, bn), :]
        h = jnp.maximum(_bn(v, st2[...], g2_ref[...], bb2_ref[...], nf), 0.0)
        for t in range(hn_ref.shape[0]):
            hn_ref[t, ...] = h[:, t * hq:(t + 1) * hq]


def _layer_last_body(hin_ref, agg_ref, bat_ref, e1p_ref, w1_ref, b1_ref,
                     g1_ref, bb1_ref, w2_ref, b2_ref, g2_ref, bb2_ref,
                     wq1_ref, bq1_ref, wq2_ref, bq2_ref, q_ref,
                     s_s, st1, st2, pool_s, cnt_s, ge_s, *,
                     nf, bn, n_graphs):
    p = pl.program_id(0)
    i = pl.program_id(1)
    _gin_phases01(p, i, bn, hin_ref, agg_ref, e1p_ref, w1_ref, b1_ref,
                  g1_ref, bb1_ref, w2_ref, b2_ref, s_s, st1, st2,
                  nf=nf, first=False)

    bids = bat_ref[...]
    lanes = jax.lax.broadcasted_iota(jnp.int32, (bn, n_graphs), 1)
    onehot = (bids == lanes).astype(jnp.float32)

    def _h():
        v = s_s[pl.ds(i * bn, bn), :]
        return jnp.maximum(_bn(v, st2[...], g2_ref[...], bb2_ref[...], nf),
                           0.0)

    @pl.when(p == 2)
    def _():
        h = _h()

        @pl.when(i == 0)
        def _():
            pool_s[...] = jnp.zeros_like(pool_s)
            cnt_s[...] = jnp.zeros_like(cnt_s)

        pool_s[...] += _dot_t(onehot, h)
        cnt_s[...] += _dot_t(onehot, jnp.ones((bn, 1), jnp.float32))

        @pl.when(i == pl.num_programs(1) - 1)
        def _():
            cnt = jnp.maximum(cnt_s[...], 1.0)
            ge_s[...] = pool_s[...] / cnt

    @pl.when(p == 3)
    def _():
        h = _h()
        gepn = _dot(onehot, ge_s[...])
        cc = jnp.concatenate([h, gepn], axis=1)
        r = jnp.maximum(_dot(cc, wq1_ref[...]) + bq1_ref[...], 0.0)
        q = jnp.sum(r * wq2_ref[...], axis=1, keepdims=True) + bq2_ref[...]
        q_ref[...] = q


def _row_pspec(bn, w, on_phase):
    return pl.BlockSpec(
        (bn, w), lambda p, i: (jnp.where(p == on_phase, i, 0), 0))


def _grp_pspec(ng, bn, w, on_phase):
    return pl.BlockSpec(
        (ng, bn, w), lambda p, i: (0, jnp.where(p == on_phase, i, 0), 0))


def _const_pspec(shape):
    return pl.BlockSpec(shape, lambda p, i: tuple(0 for _ in shape))


def _bat_pspec(bn):
    return pl.BlockSpec((bn, 1), lambda p, i: (i, 0))


def kernel(x, edge_index, batch, params):
    n = x.shape[0]
    in_dim = x.shape[1]
    w8 = 8
    h_dim = params['layers'][0]['W1'].shape[1]
    hq = h_dim // 4
    n_graphs = params['Wq1'].shape[0] - h_dim
    bn = 2000
    assert n % bn == 0
    grid = n // bn

    e = edge_index.shape[1]
    r_pad = -((-e) // (128 * _SC_TILES * _SC_G)) * (_SC_TILES * _SC_G)
    e_pad = r_pad * 128 - e
    acc_n = -((-(n + 128)) // (_SC_TILES * 8)) * (_SC_TILES * 8)

    src = edge_index[0].astype(jnp.int32)
    dst = edge_index[1].astype(jnp.int32)
    pad_ids = jnp.arange(e_pad, dtype=jnp.int32)
    srcr = jnp.concatenate([src, pad_ids % n]).reshape(r_pad, 128)
    dstr = jnp.concatenate([dst, n + (pad_ids % 128)]).reshape(r_pad, 128)
    zeros1 = jnp.zeros((acc_n // _SC_TILES, w8), jnp.float32)
    zeros4 = jnp.zeros((acc_n // _SC_TILES, hq), jnp.float32)
    sc_scatter1 = _make_sc_scatter1(n, w8, r_pad, acc_n)
    sc_scatter4 = _make_sc_scatter4(n, hq, r_pad, acc_n)

    x_pad = jnp.pad(x, ((0, 0), (0, w8 - in_dim)))
    bat2d = batch.reshape(n, 1).astype(jnp.int32)
    layers = params['layers']
    nf = float(n)

    def lay_consts(lp, w1):
        return [
            (1.0 + lp['eps']).reshape(1, 1).astype(jnp.float32), w1,
            lp['b1'].reshape(1, h_dim),
            lp['bn1_g'].reshape(1, h_dim), lp['bn1_b'].reshape(1, h_dim),
            lp['W2'], lp['b2'].reshape(1, h_dim),
            lp['obn_g'].reshape(1, h_dim), lp['obn_b'].reshape(1, h_dim),
        ]

    def lay_const_specs(k1):
        return [
            _const_pspec((1, 1)), _const_pspec((k1, h_dim)),
            _const_pspec((1, h_dim)),
            _const_pspec((1, h_dim)), _const_pspec((1, h_dim)),
            _const_pspec((h_dim, h_dim)), _const_pspec((1, h_dim)),
            _const_pspec((1, h_dim)), _const_pspec((1, h_dim)),
        ]

    mid_scratch = [
        pltpu.VMEM((n, h_dim), jnp.float32),
        pltpu.VMEM((2, h_dim), jnp.float32),
        pltpu.VMEM((2, h_dim), jnp.float32),
    ]

    h = None
    for li, lp in enumerate(layers):
        last = li + 1 == len(layers)
        if li == 0:
            agg = sc_scatter1(x_pad, srcr, dstr, zeros1)
            w1 = jnp.pad(lp['W1'], ((0, w8 - in_dim), (0, 0)))
            hin, hin_spec = x_pad, _row_pspec(bn, w8, 0)
            agg_spec = _grp_pspec(2, bn, w8, 0)
            k1, first = w8, True
        else:
            agg = sc_scatter4(h, srcr, dstr, zeros4)
            w1 = lp['W1']
            hin, hin_spec = h, _grp_pspec(4, bn, hq, 0)
            agg_spec = _grp_pspec(4, bn, hq, 0)
            k1, first = h_dim, False

        if not last:
            h = pl.pallas_call(
                functools.partial(_layer_mid_body, nf=nf, hq=hq, bn=bn,
                                  first=first),
                grid=(3, grid),
                in_specs=[hin_spec, agg_spec] + lay_const_specs(k1),
                out_specs=_grp_pspec(4, bn, hq, 2),
                out_shape=jax.ShapeDtypeStruct((4, n, hq), jnp.float32),
                scratch_shapes=mid_scratch,
                compiler_params=pltpu.CompilerParams(
                    vmem_limit_bytes=100 * 1024 * 1024),
            )(hin, agg, *lay_consts(lp, w1))
        else:
            q = pl.pallas_call(
                functools.partial(_layer_last_body, nf=nf, bn=bn,
                                  n_graphs=n_graphs),
                grid=(4, grid),
                in_specs=[hin_spec, agg_spec, _bat_pspec(bn)]
                + lay_const_specs(k1)
                + [_const_pspec((2 * h_dim, h_dim)), _const_pspec((1, h_dim)),
                   _const_pspec((1, h_dim)), _const_pspec((1, 1))],
                out_specs=_row_pspec(bn, 1, 3),
                out_shape=jax.ShapeDtypeStruct((n, 1), jnp.float32),
                scratch_shapes=[
                    pltpu.VMEM((n, h_dim), jnp.float32),
                    pltpu.VMEM((2, h_dim), jnp.float32),
                    pltpu.VMEM((2, h_dim), jnp.float32),
                    pltpu.VMEM((n_graphs, h_dim), jnp.float32),
                    pltpu.VMEM((n_graphs, 1), jnp.float32),
                    pltpu.VMEM((n_graphs, h_dim), jnp.float32),
                ],
                compiler_params=pltpu.CompilerParams(
                    vmem_limit_bytes=100 * 1024 * 1024),
            )(hin, agg, bat2d, *lay_consts(lp, w1), params['Wq1'],
              params['bq1'].reshape(1, h_dim), params['Wq2'].reshape(1, h_dim),
              params['bq2'].reshape(1, 1))

    return q.reshape(n)
